# baseline (speedup 1.0000x reference)
import functools

import jax
import jax.numpy as jnp
from jax import lax
from jax.experimental import pallas as pl
from jax.experimental.pallas import tpu as pltpu
from jax.experimental.pallas import tpu_sc as plsc

_NC = 2
_NS = 16
_NW = _NC * _NS


@functools.partial(jax.jit, static_argnums=(2, 3, 4))
def _sc_gather(table, idx, b_per_w, chunk, n_chunks):
    V, D = table.shape
    B = idx.shape[0]
    mesh = plsc.VectorSubcoreMesh(core_axis_name="c", subcore_axis_name="s")

    @functools.partial(
        pl.kernel,
        out_type=jax.ShapeDtypeStruct((B, D), jnp.float32),
        mesh=mesh,
        scratch_types=[
            pltpu.VMEM((b_per_w,), jnp.int32),
            pltpu.VMEM((chunk, D), jnp.float32),
            pltpu.SemaphoreType.DMA,
        ],
        compiler_params=pltpu.CompilerParams(use_tc_tiling_on_sc=False),
    )
    def k(table_hbm, idx_hbm, out_hbm, idx_v, rows_v, sem):
        wid = lax.axis_index("s") * _NC + lax.axis_index("c")
        base = wid * b_per_w
        pltpu.sync_copy(idx_hbm.at[pl.ds(base, b_per_w)], idx_v)

        def body(i, _):
            off = i * chunk
            pltpu.async_copy(
                table_hbm.at[idx_v.at[pl.ds(off, chunk)]], rows_v, sem
            ).wait()
            pltpu.sync_copy(rows_v, out_hbm.at[pl.ds(base + off, chunk)])
            return ()

        lax.fori_loop(0, n_chunks, body, ())

    return k(table, idx)


def kernel(x, token_embedding_table):
    Bx, S = x.shape
    V, D = token_embedding_table.shape
    B = Bx * S
    b_per_w = B // _NW
    chunk = 64
    n_chunks = b_per_w // chunk
    flat = x.reshape(B).astype(jnp.int32)
    out = _sc_gather(token_embedding_table, flat, b_per_w, chunk, n_chunks)
    return out.reshape(Bx, S, D)


# speedup vs baseline: 1.0102x; 1.0102x over previous
import functools

import jax
import jax.numpy as jnp
from jax import lax
from jax.experimental import pallas as pl
from jax.experimental.pallas import tpu as pltpu
from jax.experimental.pallas import tpu_sc as plsc

_NC = 2
_NS = 16
_NW = _NC * _NS


_NBUF = 2


@functools.partial(jax.jit, static_argnums=(2, 3, 4))
def _sc_gather(table, idx, b_per_w, chunk, n_outer):
    V, D = table.shape
    B = idx.shape[0]
    mesh = plsc.VectorSubcoreMesh(core_axis_name="c", subcore_axis_name="s")

    @functools.partial(
        pl.kernel,
        out_type=jax.ShapeDtypeStruct((B, D), jnp.float32),
        mesh=mesh,
        scratch_types=[
            pltpu.VMEM((b_per_w,), jnp.int32),
            [pltpu.VMEM((chunk, D), jnp.float32) for _ in range(_NBUF)],
            [pltpu.SemaphoreType.DMA for _ in range(_NBUF)],
            [pltpu.SemaphoreType.DMA for _ in range(_NBUF)],
        ],
        compiler_params=pltpu.CompilerParams(use_tc_tiling_on_sc=False),
    )
    def k(table_hbm, idx_hbm, out_hbm, idx_v, bufs, semg, semw):
        wid = lax.axis_index("s") * _NC + lax.axis_index("c")
        base = wid * b_per_w
        pltpu.sync_copy(idx_hbm.at[pl.ds(base, b_per_w)], idx_v)

        def gather_start(g, b):
            off = g * chunk
            pltpu.make_async_copy(
                table_hbm.at[idx_v.at[pl.ds(off, chunk)]], bufs[b], semg[b]
            ).start()

        def gather_wait(b):
            pltpu.make_async_copy(
                table_hbm.at[idx_v.at[pl.ds(0, chunk)]], bufs[b], semg[b]
            ).wait()

        def write_start(g, b):
            off = g * chunk
            pltpu.make_async_copy(
                bufs[b], out_hbm.at[pl.ds(base + off, chunk)], semw[b]
            ).start()

        def write_wait(b):
            pltpu.make_async_copy(
                bufs[b], out_hbm.at[pl.ds(base, chunk)], semw[b]
            ).wait()

        for b in range(_NBUF):
            gather_start(b, b)

        def body(j, _):
            for b in range(_NBUF):
                gather_wait(b)
                write_start(j * _NBUF + b, b)
            for b in range(_NBUF):
                write_wait(b)

                @pl.when(j < n_outer - 1)
                def _():
                    gather_start((j + 1) * _NBUF + b, b)

            return ()

        lax.fori_loop(0, n_outer, body, ())

    return k(table, idx)


def kernel(x, token_embedding_table):
    Bx, S = x.shape
    V, D = token_embedding_table.shape
    B = Bx * S
    b_per_w = B // _NW
    chunk = 40
    n_outer = b_per_w // (chunk * _NBUF)
    flat = x.reshape(B).astype(jnp.int32)
    out = _sc_gather(token_embedding_table, flat, b_per_w, chunk, n_outer)
    return out.reshape(Bx, S, D)


# speedup vs baseline: 1.7688x; 1.7510x over previous
import functools

import jax
import jax.numpy as jnp
from jax import lax
from jax.experimental import pallas as pl
from jax.experimental.pallas import tpu as pltpu
from jax.experimental.pallas import tpu_sc as plsc

_NC = 2
_NS = 16
_NW = _NC * _NS
_NBUF = 2


@functools.partial(jax.jit, static_argnums=(2, 3, 4))
def _sc_gather(table, idx, b_per_w, chunk, n_outer):
    V, D = table.shape
    B = idx.shape[0]
    mesh = plsc.VectorSubcoreMesh(core_axis_name="c", subcore_axis_name="s")

    @functools.partial(
        pl.kernel,
        out_type=jax.ShapeDtypeStruct((B, D), jnp.float32),
        mesh=mesh,
        scratch_types=[
            pltpu.VMEM((b_per_w,), jnp.int32),
            [pltpu.VMEM((chunk, D), jnp.float32) for _ in range(_NBUF)],
            [pltpu.SemaphoreType.DMA for _ in range(_NBUF)],
            [pltpu.SemaphoreType.DMA for _ in range(_NBUF)],
        ],
    )
    def k(table_hbm, idx_hbm, out_hbm, idx_v, bufs, semg, semw):
        wid = lax.axis_index("s") * _NC + lax.axis_index("c")
        base = wid * b_per_w
        pltpu.sync_copy(idx_hbm.at[pl.ds(base, b_per_w)], idx_v)

        def gather_start(g, b):
            off = g * chunk
            for q in range(chunk // 16):
                vec = idx_v[pl.ds(off + q * 16, 16)]
                for r in range(16):
                    pltpu.make_async_copy(
                        table_hbm.at[pl.ds(vec[r], 1), :],
                        bufs[b].at[pl.ds(q * 16 + r, 1), :],
                        semg[b],
                    ).start()

        def gather_wait(b):
            pltpu.make_async_copy(
                table_hbm.at[pl.ds(0, chunk), :], bufs[b], semg[b]
            ).wait()

        def write_start(g, b):
            off = g * chunk
            pltpu.make_async_copy(
                bufs[b], out_hbm.at[pl.ds(base + off, chunk)], semw[b]
            ).start()

        def write_wait(b):
            pltpu.make_async_copy(
                bufs[b], out_hbm.at[pl.ds(base, chunk)], semw[b]
            ).wait()

        for b in range(_NBUF):
            gather_start(b, b)

        def body(j, _):
            for b in range(_NBUF):
                gather_wait(b)
                write_start(j * _NBUF + b, b)

            for b in range(_NBUF):
                write_wait(b)

                @pl.when(j < n_outer - 1)
                def _():
                    gather_start((j + 1) * _NBUF + b, b)

            return ()

        lax.fori_loop(0, n_outer, body, ())

    return k(table, idx)


def kernel(x, token_embedding_table):
    Bx, S = x.shape
    V, D = token_embedding_table.shape
    B = Bx * S
    b_per_w = B // _NW
    chunk = 32
    n_outer = b_per_w // (chunk * _NBUF)
    flat = x.reshape(B).astype(jnp.int32)
    out = _sc_gather(token_embedding_table, flat, b_per_w, chunk, n_outer)
    return out.reshape(Bx, S, D)
